# baseline (device time: 180251 ns/iter reference)
import jax
import jax.numpy as jnp
from jax import lax
from jax.experimental import pallas as pl
from jax.experimental.pallas import tpu as pltpu

N_DEV = 4
SQ = 1024
SKV = 1024
NH = 32
NH_LOC = 8
DH = 128
DM = 1024
SCALE = 0.08838834764831843


def _prep_kv(k, v):
    def body(k_ref, v_ref, kt_ref, vt_ref):
        for head in range(NH):
            sl = slice(head * DH, (head + 1) * DH)
            kt_ref[head] = k_ref[:, sl].astype(jnp.bfloat16)
            vt_ref[head] = v_ref[:, sl].astype(jnp.bfloat16)

    return pl.pallas_call(
        body,
        out_shape=(
            jax.ShapeDtypeStruct((NH, SKV, DH), jnp.bfloat16),
            jax.ShapeDtypeStruct((NH, SKV, DH), jnp.bfloat16),
        ),
        in_specs=[pl.BlockSpec(memory_space=pltpu.VMEM)] * 2,
        out_specs=(pl.BlockSpec(memory_space=pltpu.VMEM),) * 2,
    )(k, v)


def _prep_xw(x2, Wq, Wo):
    def body(x_ref, wq_ref, wo_ref, xb_ref, wqb_ref, wob_ref):
        xb_ref[...] = (x_ref[...] * SCALE).astype(jnp.bfloat16)
        wqb_ref[...] = wq_ref[...].astype(jnp.bfloat16)
        wob_ref[...] = wo_ref[...].astype(jnp.bfloat16)

    return pl.pallas_call(
        body,
        out_shape=(
            jax.ShapeDtypeStruct((SQ, DM), jnp.bfloat16),
            jax.ShapeDtypeStruct((DM, DM), jnp.bfloat16),
            jax.ShapeDtypeStruct((DM, DM), jnp.bfloat16),
        ),
        in_specs=[pl.BlockSpec(memory_space=pltpu.VMEM)] * 3,
        out_specs=(pl.BlockSpec(memory_space=pltpu.VMEM),) * 3,
    )(x2, Wq, Wo)


def kernel(x, Wq, K_ext, V_ext, Wo):
    kt, vt = _prep_kv(K_ext.reshape(SKV, NH * DH), V_ext.reshape(SKV, NH * DH))
    xb, wq, wo = _prep_xw(x.reshape(SQ, DM), Wq, Wo)

    def body(x_ref, wq_ref, k_ref, v_ref, wo_ref, out_ref,
             wq_ring, wo_ring, bias_ref, ctx_ref,
             cw_send, cw_recv, ccw_send, ccw_recv):
        my = lax.axis_index("i")
        right = lax.rem(my + 1, N_DEV)
        left = lax.rem(my + N_DEV - 1, N_DEV)

        barrier_sem = pltpu.get_barrier_semaphore()
        for nbr in (left, right):
            pl.semaphore_signal(
                barrier_sem, inc=1,
                device_id=(nbr,), device_id_type=pl.DeviceIdType.MESH,
            )
        pl.semaphore_wait(barrier_sem, 2)

        rows = lax.broadcasted_iota(jnp.int32, (SQ, SKV), 0)
        cols = lax.broadcasted_iota(jnp.int32, (SQ, SKV), 1)
        qb = (rows + my * SQ) // 64
        kb = cols // 64
        mask = (qb == kb) | (kb == 0) | (lax.rem(qb + kb, 3) == 0)
        bias_ref[...] = jnp.where(mask, -20.0, -1e9).astype(jnp.bfloat16)

        def attn(h, wq_src, wo_src, first=False):
            j = lax.rem(my + N_DEV - h, N_DEV)
            q = jnp.dot(x_ref[...], wq_src[...],
                        preferred_element_type=jnp.float32)
            qb16 = q.astype(jnp.bfloat16)
            for hh in range(NH_LOC):
                head = j * NH_LOC + hh
                s = lax.dot_general(
                    qb16[:, hh * DH:(hh + 1) * DH], k_ref[head],
                    (((1,), (1,)), ((), ())),
                    preferred_element_type=jnp.float32)
                w = jnp.exp(s + bias_ref[...].astype(jnp.float32))
                denom = jnp.sum(w, axis=1, keepdims=True)
                ctxh = jnp.dot(w.astype(jnp.bfloat16), v_ref[head],
                               preferred_element_type=jnp.float32)
                ctxb = (ctxh / denom).astype(jnp.bfloat16)
                if wo_src is None:
                    ctx_ref[:, hh * DH:(hh + 1) * DH] = ctxb
                else:
                    contrib = jnp.dot(
                        ctxb, wo_src[hh * DH:(hh + 1) * DH, :],
                        preferred_element_type=jnp.float32)
                    if first and hh == 0:
                        out_ref[...] = contrib
                    else:
                        out_ref[...] += contrib

        for h in range(N_DEV - 1):
            cw = pltpu.make_async_remote_copy(
                src_ref=wq_ref if h == 0 else wq_ring.at[h - 1],
                dst_ref=wq_ring.at[h],
                send_sem=cw_send.at[h], recv_sem=cw_recv.at[h],
                device_id=(right,), device_id_type=pl.DeviceIdType.MESH,
            )
            ccw = pltpu.make_async_remote_copy(
                src_ref=wo_ref if h == 0 else wo_ring.at[h - 1],
                dst_ref=wo_ring.at[h],
                send_sem=ccw_send.at[h], recv_sem=ccw_recv.at[h],
                device_id=(left,), device_id_type=pl.DeviceIdType.MESH,
            )
            cw.start()
            ccw.start()
            if h == 0:
                attn(0, wq_ref, wo_ref, first=True)
            elif h == 1:
                attn(1, wq_ring.at[0], None)
            else:
                attn(2, wq_ring.at[1], wo_ring.at[1])
            cw.wait()
            ccw.wait()

        out_ref[...] += jnp.dot(ctx_ref[...], wo_ring[2],
                                preferred_element_type=jnp.float32)
        attn(3, wq_ring.at[2], wo_ring.at[0])

    out = pl.pallas_call(
        body,
        out_shape=jax.ShapeDtypeStruct((SQ, DM), jnp.float32),
        in_specs=[pl.BlockSpec(memory_space=pltpu.VMEM)] * 5,
        out_specs=pl.BlockSpec(memory_space=pltpu.VMEM),
        scratch_shapes=[
            pltpu.VMEM((N_DEV - 1, DM, DM), jnp.bfloat16),
            pltpu.VMEM((N_DEV - 1, DM, DM), jnp.bfloat16),
            pltpu.VMEM((SQ, SKV), jnp.bfloat16),
            pltpu.VMEM((SQ, NH_LOC * DH), jnp.bfloat16),
            pltpu.SemaphoreType.DMA((N_DEV - 1,)),
            pltpu.SemaphoreType.DMA((N_DEV - 1,)),
            pltpu.SemaphoreType.DMA((N_DEV - 1,)),
            pltpu.SemaphoreType.DMA((N_DEV - 1,)),
        ],
        compiler_params=pltpu.CompilerParams(
            collective_id=0,
            vmem_limit_bytes=100 * 1024 * 1024,
        ),
    )(xb, wq, kt, vt, wo)
    return out.reshape(1, SQ, DM)


# device time: 150245 ns/iter; 1.1997x vs baseline; 1.1997x over previous
import jax
import jax.numpy as jnp
from jax import lax
from jax.experimental import pallas as pl
from jax.experimental.pallas import tpu as pltpu

N_DEV = 4
SQ = 1024
SKV = 1024
NH = 32
NH_LOC = 8
DH = 128
DM = 1024
SCALE = 0.08838834764831843


def kernel(x, Wq, K_ext, V_ext, Wo):
    xb = (x.reshape(SQ, DM) * SCALE).astype(jnp.bfloat16)
    wq = Wq.astype(jnp.bfloat16)
    wo = Wo.astype(jnp.bfloat16)
    kt = K_ext.reshape(SKV, NH, DH).astype(jnp.bfloat16).transpose(1, 0, 2)
    vt = V_ext.reshape(SKV, NH, DH).astype(jnp.bfloat16).transpose(1, 0, 2)

    def body(x_ref, wq_ref, k_ref, v_ref, wo_ref, out_ref,
             wq_ring, wo_ring, bias_ref, ctx_ref,
             cw_send, cw_recv, ccw_send, ccw_recv):
        my = lax.axis_index("i")
        right = lax.rem(my + 1, N_DEV)
        left = lax.rem(my + N_DEV - 1, N_DEV)

        barrier_sem = pltpu.get_barrier_semaphore()
        for nbr in (left, right):
            pl.semaphore_signal(
                barrier_sem, inc=1,
                device_id=(nbr,), device_id_type=pl.DeviceIdType.MESH,
            )
        pl.semaphore_wait(barrier_sem, 2)

        rows = lax.broadcasted_iota(jnp.int32, (SQ, SKV), 0)
        cols = lax.broadcasted_iota(jnp.int32, (SQ, SKV), 1)
        qb = (rows + my * SQ) // 64
        kb = cols // 64
        mask = (qb == kb) | (kb == 0) | (lax.rem(qb + kb, 3) == 0)
        bias_ref[...] = jnp.where(mask, -20.0, -1e9).astype(jnp.bfloat16)

        def attn(h, wq_src, wo_src, first=False):
            j = lax.rem(my + N_DEV - h, N_DEV)
            q = jnp.dot(x_ref[...], wq_src[...],
                        preferred_element_type=jnp.float32)
            qb16 = q.astype(jnp.bfloat16)
            for hh in range(NH_LOC):
                head = j * NH_LOC + hh
                s = lax.dot_general(
                    qb16[:, hh * DH:(hh + 1) * DH], k_ref[head],
                    (((1,), (1,)), ((), ())),
                    preferred_element_type=jnp.float32)
                w = jnp.exp(s + bias_ref[...].astype(jnp.float32))
                denom = jnp.sum(w, axis=1, keepdims=True)
                ctxh = jnp.dot(w.astype(jnp.bfloat16), v_ref[head],
                               preferred_element_type=jnp.float32)
                ctxb = (ctxh / denom).astype(jnp.bfloat16)
                if wo_src is None:
                    ctx_ref[:, hh * DH:(hh + 1) * DH] = ctxb
                else:
                    contrib = jnp.dot(
                        ctxb, wo_src[hh * DH:(hh + 1) * DH, :],
                        preferred_element_type=jnp.float32)
                    if first and hh == 0:
                        out_ref[...] = contrib
                    else:
                        out_ref[...] += contrib

        for h in range(N_DEV - 1):
            cw = pltpu.make_async_remote_copy(
                src_ref=wq_ref if h == 0 else wq_ring.at[h - 1],
                dst_ref=wq_ring.at[h],
                send_sem=cw_send.at[h], recv_sem=cw_recv.at[h],
                device_id=(right,), device_id_type=pl.DeviceIdType.MESH,
            )
            ccw = pltpu.make_async_remote_copy(
                src_ref=wo_ref if h == 0 else wo_ring.at[h - 1],
                dst_ref=wo_ring.at[h],
                send_sem=ccw_send.at[h], recv_sem=ccw_recv.at[h],
                device_id=(left,), device_id_type=pl.DeviceIdType.MESH,
            )
            cw.start()
            ccw.start()
            if h == 0:
                attn(0, wq_ref, wo_ref, first=True)
            elif h == 1:
                attn(1, wq_ring.at[0], None)
            else:
                attn(2, wq_ring.at[1], wo_ring.at[1])
            cw.wait()
            ccw.wait()

        out_ref[...] += jnp.dot(ctx_ref[...], wo_ring[2],
                                preferred_element_type=jnp.float32)
        attn(3, wq_ring.at[2], wo_ring.at[0])

    out = pl.pallas_call(
        body,
        out_shape=jax.ShapeDtypeStruct((SQ, DM), jnp.float32),
        in_specs=[pl.BlockSpec(memory_space=pltpu.VMEM)] * 5,
        out_specs=pl.BlockSpec(memory_space=pltpu.VMEM),
        scratch_shapes=[
            pltpu.VMEM((N_DEV - 1, DM, DM), jnp.bfloat16),
            pltpu.VMEM((N_DEV - 1, DM, DM), jnp.bfloat16),
            pltpu.VMEM((SQ, SKV), jnp.bfloat16),
            pltpu.VMEM((SQ, NH_LOC * DH), jnp.bfloat16),
            pltpu.SemaphoreType.DMA((N_DEV - 1,)),
            pltpu.SemaphoreType.DMA((N_DEV - 1,)),
            pltpu.SemaphoreType.DMA((N_DEV - 1,)),
            pltpu.SemaphoreType.DMA((N_DEV - 1,)),
        ],
        compiler_params=pltpu.CompilerParams(
            collective_id=0,
            vmem_limit_bytes=100 * 1024 * 1024,
        ),
    )(xb, wq, kt, vt, wo)
    return out.reshape(1, SQ, DM)
